# Initial kernel scaffold; baseline (speedup 1.0000x reference)
import jax
import jax.numpy as jnp
from jax.experimental import pallas as pl


def kernel(x_img, x_tab, w1, b1, w2, b2):
    raise NotImplementedError("write your pallas kernel here")


# speedup vs baseline: 1.1429x; 1.1429x over previous
import jax
import jax.numpy as jnp
from jax.experimental import pallas as pl
from jax.experimental.pallas import tpu as pltpu


def _daft_fused_kernel(x_ref, xt_ref, w1t_ref, b1_ref, w2t_ref, b2_ref, o_ref):
    C, S = x_ref.shape
    bidx = pl.program_id(0)
    x = x_ref[...]
    pooled = jnp.sum(x, axis=1, keepdims=True) * (1.0 / S)
    lane = jax.lax.broadcasted_iota(jnp.int32, xt_ref.shape, 1)
    xt_col = jnp.sum(jnp.where(lane == bidx, xt_ref[...], 0.0),
                     axis=1, keepdims=True)
    z = jnp.concatenate([pooled, xt_col], axis=0)
    h = jax.lax.dot_general(w1t_ref[...], z, (((1,), (0,)), ((), ())),
                            preferred_element_type=jnp.float32)
    h = jnp.maximum(h + b1_ref[...], 0.0)
    y = jax.lax.dot_general(w2t_ref[...], h, (((1,), (0,)), ((), ())),
                            preferred_element_type=jnp.float32)
    y = y + b2_ref[...]
    a = y[:C, :]
    b = y[C:, :]
    o_ref[...] = a * x + b


def kernel(x_img, x_tab, w1, b1, w2, b2):
    B, C, D, H, W = x_img.shape
    S = D * H * W
    P = x_tab.shape[1]
    hidden = w1.shape[1]

    x3 = x_img.reshape(B, C, S)
    xt = x_tab.astype(jnp.float32).T
    w1t = w1.astype(jnp.float32).T
    b1c = b1.astype(jnp.float32).reshape(hidden, 1)
    w2t = w2.astype(jnp.float32).T
    b2c = b2.astype(jnp.float32).reshape(2 * C, 1)

    out = pl.pallas_call(
        _daft_fused_kernel,
        out_shape=jax.ShapeDtypeStruct((B, C, S), x_img.dtype),
        grid=(B,),
        in_specs=[
            pl.BlockSpec((pl.Squeezed(), C, S), lambda b: (b, 0, 0)),
            pl.BlockSpec((P, B), lambda b: (0, 0)),
            pl.BlockSpec((hidden, C + P), lambda b: (0, 0)),
            pl.BlockSpec((hidden, 1), lambda b: (0, 0)),
            pl.BlockSpec((2 * C, hidden), lambda b: (0, 0)),
            pl.BlockSpec((2 * C, 1), lambda b: (0, 0)),
        ],
        out_specs=pl.BlockSpec((pl.Squeezed(), C, S), lambda b: (b, 0, 0)),
        compiler_params=pltpu.CompilerParams(
            dimension_semantics=("parallel",)),
    )(x3, xt, w1t, b1c, w2t, b2c)

    return out.reshape(B, C, D, H, W)
